# Initial kernel scaffold; baseline (speedup 1.0000x reference)
import jax
import jax.numpy as jnp
from jax.experimental import pallas as pl


def kernel(node_feat, edge_feat, edge_index, batch_index, num_sampled_nodes_per_hop, num_sampled_edges_per_hop, W_net, b_net, W_root, root_bias, bn_gamma, bn_beta):
    raise NotImplementedError("write your pallas kernel here")


# speedup vs baseline: 3.3723x; 3.3723x over previous
import functools

import jax
import jax.numpy as jnp
from jax import lax
from jax.experimental import pallas as pl
from jax.experimental.pallas import tpu as pltpu
from jax.experimental.pallas import tpu_sc as plsc

N = 10000
E = 160000
IN_DIM = 16
OUT_DIM = 16
EDGE_DIM = 16

NC = 2
NS = 16
NW = NC * NS
CHUNK = 128
CH_PER_W = 40
EPW = CHUNK * CH_PER_W
E_PAD = EPW * NW
N_SP = 10048
ZROWS = N_SP // NS
OROWS = N // NS


def _sc_gather_body(node_hbm, idx_hbm, out_hbm, idx_v, rows_v, sem):
    c = lax.axis_index("c")
    s = lax.axis_index("s")
    wid = s * NC + c
    base = wid * EPW
    pltpu.sync_copy(idx_hbm.at[pl.ds(wid * CH_PER_W, CH_PER_W)], idx_v)

    def body(j, carry):
        pltpu.async_copy(node_hbm.at[idx_v.at[j]], rows_v, sem).wait()
        pltpu.sync_copy(rows_v, out_hbm.at[pl.ds(base + j * CHUNK, CHUNK)])
        return carry

    lax.fori_loop(0, CH_PER_W, body, 0)


@functools.lru_cache(maxsize=None)
def _sc_gather():
    mesh = plsc.VectorSubcoreMesh(
        core_axis_name="c", subcore_axis_name="s", num_cores=NC, num_subcores=NS
    )
    return pl.kernel(
        _sc_gather_body,
        out_type=jax.ShapeDtypeStruct((E_PAD, IN_DIM), jnp.float32),
        mesh=mesh,
        compiler_params=pltpu.CompilerParams(use_tc_tiling_on_sc=False),
        scratch_types=[
            pltpu.VMEM((CH_PER_W, CHUNK), jnp.int32),
            pltpu.VMEM((CHUNK, IN_DIM), jnp.float32),
            pltpu.SemaphoreType.DMA,
        ],
    )


def _sc_scatter_body(msg_hbm, idx_hbm, agg_out, cnt_out,
                     idx_v, val_v, ones_v, zero_v, agg_sh, cnt_sh):
    c = lax.axis_index("c")
    s = lax.axis_index("s")
    wid = s * NC + c

    def zrow(i, carry):
        zero_v[i, :] = jnp.zeros((16,), jnp.float32)
        return carry

    lax.fori_loop(0, ZROWS, zrow, 0)

    def orow(i, carry):
        ones_v[i, :] = jnp.ones((16,), jnp.float32)
        return carry

    lax.fori_loop(0, CHUNK, orow, 0)

    pltpu.sync_copy(zero_v, agg_sh.at[pl.ds(s * ZROWS, ZROWS)])
    pltpu.sync_copy(zero_v, cnt_sh.at[pl.ds(s * ZROWS, ZROWS)])
    pltpu.sync_copy(idx_hbm.at[pl.ds(wid * CH_PER_W, CH_PER_W)], idx_v)
    plsc.subcore_barrier()

    def body(j, carry):
        pltpu.sync_copy(msg_hbm.at[pl.ds(wid * EPW + j * CHUNK, CHUNK)], val_v)
        pltpu.sync_copy(val_v, agg_sh.at[idx_v.at[j]], add=True)
        pltpu.sync_copy(ones_v, cnt_sh.at[idx_v.at[j]], add=True)
        return carry

    lax.fori_loop(0, CH_PER_W, body, 0)
    plsc.subcore_barrier()

    pltpu.sync_copy(agg_sh.at[pl.ds(s * OROWS, OROWS)],
                    agg_out.at[c, pl.ds(s * OROWS, OROWS)])
    pltpu.sync_copy(cnt_sh.at[pl.ds(s * OROWS, OROWS)],
                    cnt_out.at[c, pl.ds(s * OROWS, OROWS)])


@functools.lru_cache(maxsize=None)
def _sc_scatter():
    mesh = plsc.VectorSubcoreMesh(
        core_axis_name="c", subcore_axis_name="s", num_cores=NC, num_subcores=NS
    )
    return pl.kernel(
        _sc_scatter_body,
        out_type=(
            jax.ShapeDtypeStruct((NC, N, OUT_DIM), jnp.float32),
            jax.ShapeDtypeStruct((NC, N, OUT_DIM), jnp.float32),
        ),
        mesh=mesh,
        compiler_params=pltpu.CompilerParams(use_tc_tiling_on_sc=False),
        scratch_types=[
            pltpu.VMEM((CH_PER_W, CHUNK), jnp.int32),
            pltpu.VMEM((CHUNK, OUT_DIM), jnp.float32),
            pltpu.VMEM((CHUNK, OUT_DIM), jnp.float32),
            pltpu.VMEM((ZROWS, OUT_DIM), jnp.float32),
            pltpu.VMEM_SHARED((N_SP, OUT_DIM), jnp.float32),
            pltpu.VMEM_SHARED((N_SP, OUT_DIM), jnp.float32),
        ],
    )


MSG_BLK = 2048


def _msg_body(ef_ref, x_ref, wn_ref, bn_ref, out_ref):
    ef = ef_ref[...]
    x = x_ref[...]
    t = jnp.tanh(
        jnp.dot(ef, wn_ref[...], preferred_element_type=jnp.float32) + bn_ref[...]
    )
    i0 = lax.broadcasted_iota(jnp.int32, (IN_DIM, IN_DIM * OUT_DIM), 0)
    i1 = lax.broadcasted_iota(jnp.int32, (IN_DIM, IN_DIM * OUT_DIM), 1)
    rmat = (i1 // OUT_DIM == i0).astype(jnp.float32)
    xb = jnp.dot(x, rmat, preferred_element_type=jnp.float32)
    j0 = lax.broadcasted_iota(jnp.int32, (IN_DIM * OUT_DIM, OUT_DIM), 0)
    j1 = lax.broadcasted_iota(jnp.int32, (IN_DIM * OUT_DIM, OUT_DIM), 1)
    smat = (j0 % OUT_DIM == j1).astype(jnp.float32)
    out_ref[...] = jnp.dot(xb * t, smat, preferred_element_type=jnp.float32)


def _msg_call(ef_p, x_src, w_net, b_net2d):
    return pl.pallas_call(
        _msg_body,
        grid=(E_PAD // MSG_BLK,),
        in_specs=[
            pl.BlockSpec((MSG_BLK, EDGE_DIM), lambda i: (i, 0)),
            pl.BlockSpec((MSG_BLK, IN_DIM), lambda i: (i, 0)),
            pl.BlockSpec((EDGE_DIM, IN_DIM * OUT_DIM), lambda i: (0, 0)),
            pl.BlockSpec((1, IN_DIM * OUT_DIM), lambda i: (0, 0)),
        ],
        out_specs=pl.BlockSpec((MSG_BLK, OUT_DIM), lambda i: (i, 0)),
        out_shape=jax.ShapeDtypeStruct((E_PAD, OUT_DIM), jnp.float32),
    )(ef_p, x_src, w_net, b_net2d)


def _final_body(nf_ref, agg_ref, cnt_ref, wr_ref, rb_ref, g_ref, b_ref, out_ref):
    nf = nf_ref[...]
    agg = agg_ref[0] + agg_ref[1]
    cnt = cnt_ref[0] + cnt_ref[1]
    agg = agg / jnp.maximum(cnt, 1.0)
    out = (
        jnp.dot(nf, wr_ref[...], preferred_element_type=jnp.float32)
        + agg
        + rb_ref[...]
    )
    mu = jnp.mean(out, axis=0, keepdims=True)
    var = jnp.mean((out - mu) ** 2, axis=0, keepdims=True)
    out = (out - mu) / jnp.sqrt(var + 1e-5) * g_ref[...] + b_ref[...]
    out_ref[...] = jnp.where(out >= 0.0, out, 0.01 * out)


def _final_call(node_feat, agg_parts, cnt_parts, w_root, rb2d, g2d, b2d):
    return pl.pallas_call(
        _final_body,
        out_shape=jax.ShapeDtypeStruct((N, OUT_DIM), jnp.float32),
    )(node_feat, agg_parts, cnt_parts, w_root, rb2d, g2d, b2d)


def kernel(node_feat, edge_feat, edge_index, batch_index,
           num_sampled_nodes_per_hop, num_sampled_edges_per_hop,
           W_net, b_net, W_root, root_bias, bn_gamma, bn_beta):
    src = edge_index[0]
    dst = edge_index[1]
    pad = E_PAD - E
    src_p = jnp.pad(src, (0, pad)).reshape(NW * CH_PER_W, CHUNK)
    dst_p = jnp.pad(dst, (0, pad), constant_values=N).reshape(NW * CH_PER_W, CHUNK)
    ef_p = jnp.pad(edge_feat, ((0, pad), (0, 0)))

    x_src = _sc_gather()(node_feat, src_p)
    msg = _msg_call(ef_p, x_src, W_net, b_net.reshape(1, IN_DIM * OUT_DIM))
    agg_parts, cnt_parts = _sc_scatter()(msg, dst_p)
    out = _final_call(
        node_feat, agg_parts, cnt_parts, W_root,
        root_bias.reshape(1, OUT_DIM), bn_gamma.reshape(1, OUT_DIM),
        bn_beta.reshape(1, OUT_DIM),
    )
    return (out, edge_index, edge_feat)


# speedup vs baseline: 3.7989x; 1.1265x over previous
import functools

import jax
import jax.numpy as jnp
from jax import lax
from jax.experimental import pallas as pl
from jax.experimental.pallas import tpu as pltpu
from jax.experimental.pallas import tpu_sc as plsc

N = 10000
E = 160000
IN_DIM = 16
OUT_DIM = 16
EDGE_DIM = 16

NC = 2
NS = 16
NW = NC * NS
CHUNK = 128
CH_PER_W = 40
EPW = CHUNK * CH_PER_W
E_PAD = EPW * NW
N_SP = 10048
ZROWS = N_SP // NS
OROWS = N // NS


def _sc_gather_body(node_hbm, idx_hbm, out_hbm, idx_v, rows_v, sem):
    c = lax.axis_index("c")
    s = lax.axis_index("s")
    wid = s * NC + c
    base = wid * EPW
    pltpu.sync_copy(idx_hbm.at[pl.ds(base, EPW)], idx_v)
    pltpu.async_copy(node_hbm.at[idx_v], rows_v, sem).wait()
    pltpu.sync_copy(rows_v, out_hbm.at[pl.ds(base, EPW)])


@functools.lru_cache(maxsize=None)
def _sc_gather():
    mesh = plsc.VectorSubcoreMesh(
        core_axis_name="c", subcore_axis_name="s", num_cores=NC, num_subcores=NS
    )
    return pl.kernel(
        _sc_gather_body,
        out_type=jax.ShapeDtypeStruct((E_PAD, IN_DIM), jnp.float32),
        mesh=mesh,
        compiler_params=pltpu.CompilerParams(use_tc_tiling_on_sc=False),
        scratch_types=[
            pltpu.VMEM((EPW,), jnp.int32),
            pltpu.VMEM((EPW, IN_DIM), jnp.float32),
            pltpu.SemaphoreType.DMA,
        ],
    )


HALF = EPW // 2


def _sc_scatter_body(msg_hbm, idx_hbm, agg_out, cnt_out,
                     idx_a, idx_b, val_v, ones_v, zero_v, agg_sh, cnt_sh):
    c = lax.axis_index("c")
    s = lax.axis_index("s")
    wid = s * NC + c
    base = wid * EPW

    def zrow(i, carry):
        zero_v[i, :] = jnp.zeros((16,), jnp.float32)
        return carry

    lax.fori_loop(0, ZROWS, zrow, 0)

    def orow(i, carry):
        ones_v[i, :] = jnp.ones((16,), jnp.float32)
        return carry

    lax.fori_loop(0, HALF, orow, 0)

    pltpu.sync_copy(zero_v, agg_sh.at[pl.ds(s * ZROWS, ZROWS)])
    pltpu.sync_copy(zero_v, cnt_sh.at[pl.ds(s * ZROWS, ZROWS)])
    pltpu.sync_copy(idx_hbm.at[pl.ds(base, HALF)], idx_a)
    pltpu.sync_copy(idx_hbm.at[pl.ds(base + HALF, HALF)], idx_b)
    plsc.subcore_barrier()

    pltpu.sync_copy(msg_hbm.at[pl.ds(base, HALF)], val_v)
    pltpu.sync_copy(val_v, agg_sh.at[idx_a], add=True)
    pltpu.sync_copy(msg_hbm.at[pl.ds(base + HALF, HALF)], val_v)
    pltpu.sync_copy(val_v, agg_sh.at[idx_b], add=True)
    pltpu.sync_copy(ones_v, cnt_sh.at[idx_a], add=True)
    pltpu.sync_copy(ones_v, cnt_sh.at[idx_b], add=True)
    plsc.subcore_barrier()

    pltpu.sync_copy(agg_sh.at[pl.ds(s * OROWS, OROWS)],
                    agg_out.at[c, pl.ds(s * OROWS, OROWS)])
    pltpu.sync_copy(cnt_sh.at[pl.ds(s * OROWS, OROWS)],
                    cnt_out.at[c, pl.ds(s * OROWS, OROWS)])


@functools.lru_cache(maxsize=None)
def _sc_scatter():
    mesh = plsc.VectorSubcoreMesh(
        core_axis_name="c", subcore_axis_name="s", num_cores=NC, num_subcores=NS
    )
    return pl.kernel(
        _sc_scatter_body,
        out_type=(
            jax.ShapeDtypeStruct((NC, N, OUT_DIM), jnp.float32),
            jax.ShapeDtypeStruct((NC, N, OUT_DIM), jnp.float32),
        ),
        mesh=mesh,
        compiler_params=pltpu.CompilerParams(use_tc_tiling_on_sc=False),
        scratch_types=[
            pltpu.VMEM((HALF,), jnp.int32),
            pltpu.VMEM((HALF,), jnp.int32),
            pltpu.VMEM((HALF, OUT_DIM), jnp.float32),
            pltpu.VMEM((HALF, OUT_DIM), jnp.float32),
            pltpu.VMEM((ZROWS, OUT_DIM), jnp.float32),
            pltpu.VMEM_SHARED((N_SP, OUT_DIM), jnp.float32),
            pltpu.VMEM_SHARED((N_SP, OUT_DIM), jnp.float32),
        ],
    )


MSG_BLK = 2000


def _msg_body(ef_ref, x_ref, wn_ref, bn_ref, out_ref):
    ef = ef_ref[...]
    x = x_ref[...]
    t = jnp.tanh(
        jnp.dot(ef, wn_ref[...], preferred_element_type=jnp.float32) + bn_ref[...]
    )
    i0 = lax.broadcasted_iota(jnp.int32, (IN_DIM, IN_DIM * OUT_DIM), 0)
    i1 = lax.broadcasted_iota(jnp.int32, (IN_DIM, IN_DIM * OUT_DIM), 1)
    rmat = (i1 // OUT_DIM == i0).astype(jnp.float32)
    xb = jnp.dot(x, rmat, preferred_element_type=jnp.float32)
    j0 = lax.broadcasted_iota(jnp.int32, (IN_DIM * OUT_DIM, OUT_DIM), 0)
    j1 = lax.broadcasted_iota(jnp.int32, (IN_DIM * OUT_DIM, OUT_DIM), 1)
    smat = (j0 % OUT_DIM == j1).astype(jnp.float32)
    out_ref[...] = jnp.dot(xb * t, smat, preferred_element_type=jnp.float32)


def _msg_call(edge_feat, x_src, w_net, b_net2d):
    return pl.pallas_call(
        _msg_body,
        grid=(E // MSG_BLK,),
        in_specs=[
            pl.BlockSpec((MSG_BLK, EDGE_DIM), lambda i: (i, 0)),
            pl.BlockSpec((MSG_BLK, IN_DIM), lambda i: (i, 0)),
            pl.BlockSpec((EDGE_DIM, IN_DIM * OUT_DIM), lambda i: (0, 0)),
            pl.BlockSpec((1, IN_DIM * OUT_DIM), lambda i: (0, 0)),
        ],
        out_specs=pl.BlockSpec((MSG_BLK, OUT_DIM), lambda i: (i, 0)),
        out_shape=jax.ShapeDtypeStruct((E_PAD, OUT_DIM), jnp.float32),
    )(edge_feat, x_src, w_net, b_net2d)


def _final_body(nf_ref, agg_ref, cnt_ref, wr_ref, rb_ref, g_ref, b_ref, out_ref):
    nf = nf_ref[...]
    agg = agg_ref[0] + agg_ref[1]
    cnt = cnt_ref[0] + cnt_ref[1]
    agg = agg / jnp.maximum(cnt, 1.0)
    out = (
        jnp.dot(nf, wr_ref[...], preferred_element_type=jnp.float32)
        + agg
        + rb_ref[...]
    )
    mu = jnp.mean(out, axis=0, keepdims=True)
    var = jnp.mean((out - mu) ** 2, axis=0, keepdims=True)
    out = (out - mu) / jnp.sqrt(var + 1e-5) * g_ref[...] + b_ref[...]
    out_ref[...] = jnp.where(out >= 0.0, out, 0.01 * out)


def _final_call(node_feat, agg_parts, cnt_parts, w_root, rb2d, g2d, b2d):
    return pl.pallas_call(
        _final_body,
        out_shape=jax.ShapeDtypeStruct((N, OUT_DIM), jnp.float32),
    )(node_feat, agg_parts, cnt_parts, w_root, rb2d, g2d, b2d)


def kernel(node_feat, edge_feat, edge_index, batch_index,
           num_sampled_nodes_per_hop, num_sampled_edges_per_hop,
           W_net, b_net, W_root, root_bias, bn_gamma, bn_beta):
    src = edge_index[0]
    dst = edge_index[1]
    pad = E_PAD - E
    src_p = jnp.pad(src, (0, pad))
    dst_p = jnp.pad(dst, (0, pad), constant_values=N)

    x_src = _sc_gather()(node_feat, src_p)
    msg = _msg_call(edge_feat, x_src, W_net, b_net.reshape(1, IN_DIM * OUT_DIM))
    agg_parts, cnt_parts = _sc_scatter()(msg, dst_p)
    out = _final_call(
        node_feat, agg_parts, cnt_parts, W_root,
        root_bias.reshape(1, OUT_DIM), bn_gamma.reshape(1, OUT_DIM),
        bn_beta.reshape(1, OUT_DIM),
    )
    return (out, edge_index, edge_feat)


# speedup vs baseline: 5.1207x; 1.3479x over previous
import functools

import jax
import jax.numpy as jnp
from jax import lax
from jax.experimental import pallas as pl
from jax.experimental.pallas import tpu as pltpu
from jax.experimental.pallas import tpu_sc as plsc

N = 10000
E = 160000
IN_DIM = 16
OUT_DIM = 16
EDGE_DIM = 16

NC = 2
NS = 16
NW = NC * NS
CHUNK = 128
CH_PER_W = 40
EPW = CHUNK * CH_PER_W
E_PAD = EPW * NW
N_SP = 10048
ZROWS = N_SP // NS
OROWS = N // NS


def _sc_gather_body(node_hbm, idx_hbm, out_hbm, idx_v, rows_v, sem):
    c = lax.axis_index("c")
    s = lax.axis_index("s")
    wid = s * NC + c
    base = wid * EPW
    pltpu.sync_copy(idx_hbm.at[pl.ds(base, EPW)], idx_v)
    pltpu.async_copy(node_hbm.at[idx_v], rows_v, sem).wait()
    pltpu.sync_copy(rows_v, out_hbm.at[pl.ds(base, EPW)])


@functools.lru_cache(maxsize=None)
def _sc_gather():
    mesh = plsc.VectorSubcoreMesh(
        core_axis_name="c", subcore_axis_name="s", num_cores=NC, num_subcores=NS
    )
    return pl.kernel(
        _sc_gather_body,
        out_type=jax.ShapeDtypeStruct((E_PAD, IN_DIM), jnp.float32),
        mesh=mesh,
        compiler_params=pltpu.CompilerParams(use_tc_tiling_on_sc=False),
        scratch_types=[
            pltpu.VMEM((EPW,), jnp.int32),
            pltpu.VMEM((EPW, IN_DIM), jnp.float32),
            pltpu.SemaphoreType.DMA,
        ],
    )


HALF = EPW // 2


def _sc_scatter_body(msg_hbm, idx_hbm, agg_out, cnt_out,
                     idx_a, idx_b, val_v, ones_v, zero_v, agg_sh, cnt_sh):
    c = lax.axis_index("c")
    s = lax.axis_index("s")
    wid = s * NC + c
    base = wid * EPW

    def zrow(i, carry):
        zero_v[i, :] = jnp.zeros((16,), jnp.float32)
        return carry

    lax.fori_loop(0, ZROWS, zrow, 0)

    def orow(i, carry):
        ones_v[i, :] = jnp.ones((16,), jnp.float32)
        return carry

    lax.fori_loop(0, HALF, orow, 0)

    pltpu.sync_copy(zero_v, agg_sh.at[pl.ds(s * ZROWS, ZROWS)])
    pltpu.sync_copy(zero_v, cnt_sh.at[pl.ds(s * ZROWS, ZROWS)])
    pltpu.sync_copy(idx_hbm.at[pl.ds(base, HALF)], idx_a)
    pltpu.sync_copy(idx_hbm.at[pl.ds(base + HALF, HALF)], idx_b)
    plsc.subcore_barrier()

    pltpu.sync_copy(msg_hbm.at[pl.ds(base, HALF)], val_v)
    pltpu.sync_copy(val_v, agg_sh.at[idx_a], add=True)
    pltpu.sync_copy(msg_hbm.at[pl.ds(base + HALF, HALF)], val_v)
    pltpu.sync_copy(val_v, agg_sh.at[idx_b], add=True)
    pltpu.sync_copy(ones_v, cnt_sh.at[idx_a], add=True)
    pltpu.sync_copy(ones_v, cnt_sh.at[idx_b], add=True)
    plsc.subcore_barrier()

    pltpu.sync_copy(agg_sh.at[pl.ds(s * OROWS, OROWS)],
                    agg_out.at[c, pl.ds(s * OROWS, OROWS)])
    pltpu.sync_copy(cnt_sh.at[pl.ds(s * OROWS, OROWS)],
                    cnt_out.at[c, pl.ds(s * OROWS, OROWS)])


@functools.lru_cache(maxsize=None)
def _sc_scatter():
    mesh = plsc.VectorSubcoreMesh(
        core_axis_name="c", subcore_axis_name="s", num_cores=NC, num_subcores=NS
    )
    return pl.kernel(
        _sc_scatter_body,
        out_type=(
            jax.ShapeDtypeStruct((NC, N, OUT_DIM), jnp.float32),
            jax.ShapeDtypeStruct((NC, N, OUT_DIM), jnp.float32),
        ),
        mesh=mesh,
        compiler_params=pltpu.CompilerParams(use_tc_tiling_on_sc=False),
        scratch_types=[
            pltpu.VMEM((HALF,), jnp.int32),
            pltpu.VMEM((HALF,), jnp.int32),
            pltpu.VMEM((HALF, OUT_DIM), jnp.float32),
            pltpu.VMEM((HALF, OUT_DIM), jnp.float32),
            pltpu.VMEM((ZROWS, OUT_DIM), jnp.float32),
            pltpu.VMEM_SHARED((N_SP, OUT_DIM), jnp.float32),
            pltpu.VMEM_SHARED((N_SP, OUT_DIM), jnp.float32),
        ],
    )


PK = 128 // IN_DIM
MSG_BLK8 = 160
E8 = E // PK
E8_PAD = E_PAD // PK


def _msg_body(ef_ref, x_ref, w2_ref, b2_ref, r2_ref, s2_ref, out_ref):
    ef = ef_ref[...]
    x = x_ref[...]
    t = jnp.tanh(
        jnp.dot(ef, w2_ref[...], preferred_element_type=jnp.float32) + b2_ref[...]
    )
    xb = jnp.dot(x, r2_ref[...], preferred_element_type=jnp.float32)
    out_ref[...] = jnp.dot(xb * t, s2_ref[...], preferred_element_type=jnp.float32)


def _msg_call(ef_pk, x_pk, w2, b2, r2, s2):
    kd = PK * IN_DIM * OUT_DIM
    return pl.pallas_call(
        _msg_body,
        grid=(E8 // MSG_BLK8,),
        in_specs=[
            pl.BlockSpec((MSG_BLK8, 128), lambda i: (i, 0)),
            pl.BlockSpec((MSG_BLK8, 128), lambda i: (i, 0)),
            pl.BlockSpec((128, kd), lambda i: (0, 0)),
            pl.BlockSpec((1, kd), lambda i: (0, 0)),
            pl.BlockSpec((128, kd), lambda i: (0, 0)),
            pl.BlockSpec((kd, 128), lambda i: (0, 0)),
        ],
        out_specs=pl.BlockSpec((MSG_BLK8, 128), lambda i: (i, 0)),
        out_shape=jax.ShapeDtypeStruct((E8_PAD, 128), jnp.float32),
    )(ef_pk, x_pk, w2, b2, r2, s2)


N8 = N // PK


def _final_body(nf_ref, agg_ref, cnt_ref, wr2_ref, m_ref, rb_ref, g_ref, b_ref,
                out_ref):
    nf = nf_ref[...]
    agg = agg_ref[0] + agg_ref[1]
    cnt = cnt_ref[0] + cnt_ref[1]
    agg = agg / jnp.maximum(cnt, 1.0)
    pre = (
        jnp.dot(nf, wr2_ref[...], preferred_element_type=jnp.float32)
        + agg
        + rb_ref[...]
    )
    csum = jnp.sum(pre, axis=0, keepdims=True)
    csq = jnp.sum(pre * pre, axis=0, keepdims=True)
    mu = jnp.dot(csum, m_ref[...], preferred_element_type=jnp.float32) / N
    musq = jnp.dot(csq, m_ref[...], preferred_element_type=jnp.float32) / N
    var = musq - mu * mu
    out = (pre - mu) / jnp.sqrt(var + 1e-5) * g_ref[...] + b_ref[...]
    out_ref[...] = jnp.where(out >= 0.0, out, 0.01 * out)


def _final_call(nf_pk, agg_pk, cnt_pk, wr2, m, rb, g, b):
    return pl.pallas_call(
        _final_body,
        out_shape=jax.ShapeDtypeStruct((N8, 128), jnp.float32),
    )(nf_pk, agg_pk, cnt_pk, wr2, m, rb, g, b)


def kernel(node_feat, edge_feat, edge_index, batch_index,
           num_sampled_nodes_per_hop, num_sampled_edges_per_hop,
           W_net, b_net, W_root, root_bias, bn_gamma, bn_beta):
    src = edge_index[0]
    dst = edge_index[1]
    pad = E_PAD - E
    src_p = jnp.pad(src, (0, pad))
    dst_p = jnp.pad(dst, (0, pad), constant_values=N)

    kd = PK * IN_DIM * OUT_DIM
    eye8 = jnp.eye(PK, dtype=jnp.float32)
    w2 = jnp.kron(eye8, W_net)
    b2 = jnp.tile(b_net, PK).reshape(1, kd)
    k0 = lax.broadcasted_iota(jnp.int32, (128, kd), 0)
    c0 = lax.broadcasted_iota(jnp.int32, (128, kd), 1)
    r2 = ((k0 // IN_DIM == c0 // (IN_DIM * OUT_DIM))
          & (k0 % IN_DIM == (c0 % (IN_DIM * OUT_DIM)) // OUT_DIM)
          ).astype(jnp.float32)
    s0 = lax.broadcasted_iota(jnp.int32, (kd, 128), 0)
    s1 = lax.broadcasted_iota(jnp.int32, (kd, 128), 1)
    s2 = ((s0 // (IN_DIM * OUT_DIM) == s1 // OUT_DIM)
          & (s0 % OUT_DIM == s1 % OUT_DIM)).astype(jnp.float32)

    x_src = _sc_gather()(node_feat, src_p)
    msg = _msg_call(edge_feat.reshape(E8, 128), x_src.reshape(E8_PAD, 128),
                    w2, b2, r2, s2)
    agg_parts, cnt_parts = _sc_scatter()(msg.reshape(E_PAD, OUT_DIM), dst_p)

    wr2 = jnp.kron(eye8, W_root)
    m0 = lax.broadcasted_iota(jnp.int32, (128, 128), 0)
    m1 = lax.broadcasted_iota(jnp.int32, (128, 128), 1)
    m = (m0 % OUT_DIM == m1 % OUT_DIM).astype(jnp.float32)
    out = _final_call(
        node_feat.reshape(N8, 128), agg_parts.reshape(NC, N8, 128),
        cnt_parts.reshape(NC, N8, 128), wr2, m,
        jnp.tile(root_bias, PK).reshape(1, 128),
        jnp.tile(bn_gamma, PK).reshape(1, 128),
        jnp.tile(bn_beta, PK).reshape(1, 128),
    )
    return (out.reshape(N, OUT_DIM), edge_index, edge_feat)
